# Initial kernel scaffold; baseline (speedup 1.0000x reference)
import jax
import jax.numpy as jnp
from jax.experimental import pallas as pl


def kernel(x, edge_index, edge_attr, bc_disp, bc_rot, params):
    raise NotImplementedError("write your pallas kernel here")


# speedup vs baseline: 2.2134x; 2.2134x over previous
import functools

import jax
import jax.numpy as jnp
from jax import lax
from jax.experimental import pallas as pl
from jax.experimental.pallas import tpu as pltpu
from jax.experimental.pallas import tpu_sc as plsc

N = 10000
E = 160000
NODE_IN = 10
EDGE_IN = 7
H = 128
L = 6
OUT = 3

NC = 2
NS = 16
NW = NC * NS
CHUNK = 64
CPT = 80
E_PAD = NW * CPT * CHUNK
N_PAD = 10240
ROWS_PER_TILE = N_PAD // NS
RB_COPY = 64
DEG_W = 16
LG = H // 16

_f32 = jnp.float32


def _make_sc_scatter(layer):
    mesh = plsc.VectorSubcoreMesh(core_axis_name="c", subcore_axis_name="s")

    @functools.partial(
        pl.kernel,
        out_type=jax.ShapeDtypeStruct((NC, N_PAD, H), _f32),
        mesh=mesh,
        scratch_types=[
            pltpu.VMEM((CHUNK,), jnp.int32),
            pltpu.VMEM((CHUNK,), jnp.int32),
            pltpu.VMEM((CHUNK, H), _f32),
            pltpu.VMEM((CHUNK, H), _f32),
            pltpu.VMEM((CHUNK, H), _f32),
            pltpu.VMEM((RB_COPY, H), _f32),
            pltpu.VMEM_SHARED((N_PAD, H), _f32),
            pltpu.SemaphoreType.DMA,
            pltpu.SemaphoreType.DMA,
        ],
    )
    def sc_scatter(ps, pd, q, src, dst, out,
                   idx_s, idx_d, ba, bb, bq, stage, acc, sem1, sem2):
        cid = lax.axis_index("c")
        sid = lax.axis_index("s")
        wid = cid * NS + sid
        row0 = sid * ROWS_PER_TILE

        def zrow(r, carry):
            for g in range(LG):
                stage[r, pl.ds(g * 16, 16)] = jnp.zeros((16,), _f32)
            return carry
        lax.fori_loop(0, RB_COPY, zrow, 0)

        def zcp(t, carry):
            pltpu.sync_copy(stage, acc.at[pl.ds(row0 + t * RB_COPY, RB_COPY)])
            return carry
        lax.fori_loop(0, ROWS_PER_TILE // RB_COPY, zcp, 0)
        plsc.subcore_barrier()

        base0 = wid * (CPT * CHUNK)

        def body(j, carry):
            base = base0 + j * CHUNK
            pltpu.sync_copy(src.at[pl.ds(base, CHUNK)], idx_s)
            pltpu.sync_copy(dst.at[pl.ds(base, CHUNK)], idx_d)
            cp1 = pltpu.async_copy(ps.at[idx_s], ba, sem1)
            cp2 = pltpu.async_copy(pd.at[idx_d], bb, sem2)
            pltpu.sync_copy(q.at[layer, pl.ds(base, CHUNK)], bq)
            cp1.wait()
            cp2.wait()

            def crow(r, c2):
                for g in range(LG):
                    s = pl.ds(g * 16, 16)
                    ba[r, s] = jnp.maximum(ba[r, s] + bb[r, s] + bq[r, s],
                                           0.0)
                return c2
            lax.fori_loop(0, CHUNK, crow, 0)
            pltpu.sync_copy(ba, acc.at[idx_d], add=True)
            return carry
        lax.fori_loop(0, CPT, body, 0)
        plsc.subcore_barrier()

        def readback(t, carry):
            r = row0 + t * RB_COPY
            pltpu.sync_copy(acc.at[pl.ds(r, RB_COPY)], stage)
            pltpu.sync_copy(stage, out.at[cid, pl.ds(r, RB_COPY)])
            return carry
        lax.fori_loop(0, ROWS_PER_TILE // RB_COPY, readback, 0)

    return sc_scatter


def _make_sc_degree():
    mesh = plsc.VectorSubcoreMesh(core_axis_name="c", subcore_axis_name="s")

    @functools.partial(
        pl.kernel,
        out_type=jax.ShapeDtypeStruct((NC, N_PAD, DEG_W), _f32),
        mesh=mesh,
        scratch_types=[
            pltpu.VMEM((CHUNK,), jnp.int32),
            pltpu.VMEM((CHUNK, DEG_W), _f32),
            pltpu.VMEM((RB_COPY, DEG_W), _f32),
            pltpu.VMEM_SHARED((N_PAD, DEG_W), _f32),
        ],
    )
    def sc_degree(dst, out, idx_d, ones, stage, acc):
        cid = lax.axis_index("c")
        sid = lax.axis_index("s")
        wid = cid * NS + sid
        row0 = sid * ROWS_PER_TILE

        def fill(r, carry):
            ones[r, pl.ds(0, 16)] = jnp.ones((16,), _f32)
            stage[r, pl.ds(0, 16)] = jnp.zeros((16,), _f32)
            return carry
        lax.fori_loop(0, RB_COPY, fill, 0)

        def zcp(t, carry):
            pltpu.sync_copy(stage, acc.at[pl.ds(row0 + t * RB_COPY, RB_COPY)])
            return carry
        lax.fori_loop(0, ROWS_PER_TILE // RB_COPY, zcp, 0)
        plsc.subcore_barrier()

        base0 = wid * (CPT * CHUNK)

        def body(j, carry):
            pltpu.sync_copy(dst.at[pl.ds(base0 + j * CHUNK, CHUNK)], idx_d)
            pltpu.sync_copy(ones, acc.at[idx_d], add=True)
            return carry
        lax.fori_loop(0, CPT, body, 0)
        plsc.subcore_barrier()

        def readback(t, carry):
            r = row0 + t * RB_COPY
            pltpu.sync_copy(acc.at[pl.ds(r, RB_COPY)], stage)
            pltpu.sync_copy(stage, out.at[cid, pl.ds(r, RB_COPY)])
            return carry
        lax.fori_loop(0, ROWS_PER_TILE // RB_COPY, readback, 0)

    return sc_degree


RB_N = 2000
GRID_N = N // RB_N
EB = 2048
GRID_E = E_PAD // EB


def _dot(a, b):
    return jax.lax.dot_general(a, b, (((1,), (0,)), ((), ())),
                               preferred_element_type=_f32)


def _dot_hi(a, b):
    return jax.lax.dot_general(a, b, (((1,), (0,)), ((), ())),
                               preferred_element_type=_f32,
                               precision=jax.lax.Precision.HIGHEST)


def _tc_node_encode(x, neW0, neb0, neW1, neb1, w0s, w0d, b0):
    def body(x_ref, w0_ref, bb0_ref, w1_ref, bb1_ref, ws_ref, wd_ref, be_ref,
             h_ref, ps_ref, pd_ref):
        hb = _dot(jax.nn.relu(_dot(x_ref[...], w0_ref[...]) + bb0_ref[...]),
                  w1_ref[...]) + bb1_ref[...]
        h_ref[...] = hb
        ps_ref[...] = _dot(hb, ws_ref[...])
        pd_ref[...] = _dot(hb, wd_ref[...]) + be_ref[...]

    full = lambda r, c: pl.BlockSpec((r, c), lambda i: (0, 0))
    return pl.pallas_call(
        body,
        grid=(GRID_N,),
        in_specs=[
            pl.BlockSpec((RB_N, NODE_IN), lambda i: (i, 0)),
            full(NODE_IN, H), full(1, H), full(H, H), full(1, H),
            full(H, H), full(H, H), full(1, H),
        ],
        out_specs=[
            pl.BlockSpec((RB_N, H), lambda i: (i, 0)),
            pl.BlockSpec((RB_N, H), lambda i: (i, 0)),
            pl.BlockSpec((RB_N, H), lambda i: (i, 0)),
        ],
        out_shape=[
            jax.ShapeDtypeStruct((N, H), _f32),
            jax.ShapeDtypeStruct((N, H), _f32),
            jax.ShapeDtypeStruct((N, H), _f32),
        ],
    )(x, neW0, neb0, neW1, neb1, w0s, w0d, b0)


def _tc_edge_q(ea, eeW0, eeb0, eeW1, eeb1, w0e_all):
    def body(ea_ref, w0_ref, b0_ref, w1_ref, b1_ref, we_ref, q_ref):
        eb = _dot(jax.nn.relu(_dot(ea_ref[...], w0_ref[...]) + b0_ref[...]),
                  w1_ref[...]) + b1_ref[...]
        for l in range(L):
            q_ref[l] = _dot(eb, we_ref[l])

    full = lambda r, c: pl.BlockSpec((r, c), lambda i: (0, 0))
    return pl.pallas_call(
        body,
        grid=(GRID_E,),
        in_specs=[
            pl.BlockSpec((EB, EDGE_IN), lambda i: (i, 0)),
            full(EDGE_IN, H), full(1, H), full(H, H), full(1, H),
            pl.BlockSpec((L, H, H), lambda i: (0, 0, 0)),
        ],
        out_specs=pl.BlockSpec((L, EB, H), lambda i: (0, i, 0)),
        out_shape=jax.ShapeDtypeStruct((L, E_PAD, H), _f32),
    )(ea, eeW0, eeb0, eeW1, eeb1, w0e_all)


def _tc_layer_update(h, S, D, eW1, eb1, nW0h, nW0a, nb0, nW1, nb1,
                     w0s_n, w0d_n, b0_n):
    def body(h_ref, s_ref, d_ref, w1_ref, b1_ref, wh_ref, wa_ref, bb0_ref,
             nw1_ref, nb1_ref, ws_ref, wd_ref, be_ref,
             h_out, ps_out, pd_out):
        hb = h_ref[...]
        ssum = s_ref[0] + s_ref[1]
        deg = d_ref[0, :, 0:1] + d_ref[1, :, 0:1]
        agg = _dot(ssum, w1_ref[...]) + deg * b1_ref[...]
        t = jax.nn.relu(_dot(hb, wh_ref[...]) + _dot(agg, wa_ref[...])
                        + bb0_ref[...])
        hn = hb + _dot(t, nw1_ref[...]) + nb1_ref[...]
        h_out[...] = hn
        ps_out[...] = _dot(hn, ws_ref[...])
        pd_out[...] = _dot(hn, wd_ref[...]) + be_ref[...]

    full = lambda r, c: pl.BlockSpec((r, c), lambda i: (0, 0))
    return pl.pallas_call(
        body,
        grid=(GRID_N,),
        in_specs=[
            pl.BlockSpec((RB_N, H), lambda i: (i, 0)),
            pl.BlockSpec((NC, RB_N, H), lambda i: (0, i, 0)),
            pl.BlockSpec((NC, RB_N, DEG_W), lambda i: (0, i, 0)),
            full(H, H), full(1, H), full(H, H), full(H, H), full(1, H),
            full(H, H), full(1, H), full(H, H), full(H, H), full(1, H),
        ],
        out_specs=[
            pl.BlockSpec((RB_N, H), lambda i: (i, 0)),
            pl.BlockSpec((RB_N, H), lambda i: (i, 0)),
            pl.BlockSpec((RB_N, H), lambda i: (i, 0)),
        ],
        out_shape=[
            jax.ShapeDtypeStruct((N, H), _f32),
            jax.ShapeDtypeStruct((N, H), _f32),
            jax.ShapeDtypeStruct((N, H), _f32),
        ],
    )(h, S, D, eW1, eb1, nW0h, nW0a, nb0, nW1, nb1, w0s_n, w0d_n, b0_n)


def _tc_final(h, S, D, eW1, eb1, nW0h, nW0a, nb0, nW1, nb1,
              deW0, deb0, deW1, deb1, deW2, deb2, bc_disp, bc_rot):
    def body(h_ref, s_ref, d_ref, w1_ref, b1_ref, wh_ref, wa_ref, bb0_ref,
             nw1_ref, nb1_ref, dw0_ref, db0_ref, dw1_ref, db1_ref,
             dw2_ref, db2_ref, bcd_ref, bcr_ref, out_ref):
        hb = h_ref[...]
        ssum = s_ref[0] + s_ref[1]
        deg = d_ref[0, :, 0:1] + d_ref[1, :, 0:1]
        agg = _dot(ssum, w1_ref[...]) + deg * b1_ref[...]
        t = jax.nn.relu(_dot(hb, wh_ref[...]) + _dot(agg, wa_ref[...])
                        + bb0_ref[...])
        hn = hb + _dot(t, nw1_ref[...]) + nb1_ref[...]
        z = jax.nn.relu(_dot(hn, dw0_ref[...]) + db0_ref[...])
        z = jax.nn.relu(_dot(z, dw1_ref[...]) + db1_ref[...])
        pred = _dot(z, dw2_ref[...]) + db2_ref[...]
        mask = jnp.concatenate([1.0 - bcd_ref[...], 1.0 - bcr_ref[...]],
                               axis=1)
        out_ref[...] = pred * mask

    full = lambda r, c: pl.BlockSpec((r, c), lambda i: (0, 0))
    return pl.pallas_call(
        body,
        grid=(GRID_N,),
        in_specs=[
            pl.BlockSpec((RB_N, H), lambda i: (i, 0)),
            pl.BlockSpec((NC, RB_N, H), lambda i: (0, i, 0)),
            pl.BlockSpec((NC, RB_N, DEG_W), lambda i: (0, i, 0)),
            full(H, H), full(1, H), full(H, H), full(H, H), full(1, H),
            full(H, H), full(1, H),
            full(H, H), full(1, H), full(H, 64), full(1, 64),
            full(64, OUT), full(1, OUT),
            pl.BlockSpec((RB_N, 2), lambda i: (i, 0)),
            pl.BlockSpec((RB_N, 1), lambda i: (i, 0)),
        ],
        out_specs=pl.BlockSpec((RB_N, OUT), lambda i: (i, 0)),
        out_shape=jax.ShapeDtypeStruct((N, OUT), _f32),
    )(h, S, D, eW1, eb1, nW0h, nW0a, nb0, nW1, nb1,
      deW0, deb0, deW1, deb1, deW2, deb2, bc_disp, bc_rot)


_SC_SCATTER = [_make_sc_scatter(l) for l in range(L)]
_SC_DEGREE = _make_sc_degree()


def kernel(x, edge_index, edge_attr, bc_disp, bc_rot, params):
    p = params
    r1 = lambda v: v.reshape(1, -1)

    src = edge_index[0]
    dst = edge_index[1]
    pad = E_PAD - E
    src_p = jnp.concatenate([src, jnp.zeros((pad,), jnp.int32)])
    dst_p = jnp.concatenate([dst, jnp.full((pad,), N, jnp.int32)])
    ea_p = jnp.concatenate([edge_attr, jnp.zeros((pad, EDGE_IN), _f32)])

    w0s = [p['mp%d_eW0' % i][:H] for i in range(L)]
    w0d = [p['mp%d_eW0' % i][H:2 * H] for i in range(L)]
    w0e_all = jnp.stack([p['mp%d_eW0' % i][2 * H:] for i in range(L)])

    h, ps, pd = _tc_node_encode(
        x, p['ne_W0'], r1(p['ne_b0']), p['ne_W1'], r1(p['ne_b1']),
        w0s[0], w0d[0], r1(p['mp0_eb0']))
    q_all = _tc_edge_q(ea_p, p['ee_W0'], r1(p['ee_b0']),
                       p['ee_W1'], r1(p['ee_b1']), w0e_all)
    D = _SC_DEGREE(dst_p)

    for i in range(L):
        S = _SC_SCATTER[i](ps, pd, q_all, src_p, dst_p)
        args = (p['mp%d_eW1' % i], r1(p['mp%d_eb1' % i]),
                p['mp%d_nW0' % i][:H], p['mp%d_nW0' % i][H:],
                r1(p['mp%d_nb0' % i]), p['mp%d_nW1' % i],
                r1(p['mp%d_nb1' % i]))
        if i < L - 1:
            h, ps, pd = _tc_layer_update(
                h, S, D, *args,
                w0s[i + 1], w0d[i + 1], r1(p['mp%d_eb0' % (i + 1)]))
        else:
            pred = _tc_final(
                h, S, D, *args,
                p['de_W0'], r1(p['de_b0']), p['de_W1'], r1(p['de_b1']),
                p['de_W2'], r1(p['de_b2']), bc_disp, bc_rot)
    return pred


# speedup vs baseline: 2.3738x; 1.0725x over previous
import functools

import jax
import jax.numpy as jnp
from jax import lax
from jax.experimental import pallas as pl
from jax.experimental.pallas import tpu as pltpu
from jax.experimental.pallas import tpu_sc as plsc

N = 10000
E = 160000
NODE_IN = 10
EDGE_IN = 7
H = 128
L = 6
OUT = 3

NC = 2
NS = 16
NW = NC * NS
CHUNK = 64
CPT = 80
E_PAD = NW * CPT * CHUNK
N_PAD = 10240
ROWS_PER_TILE = N_PAD // NS
RB_COPY = 32
DEG_W = 16
LG = H // 16

_f32 = jnp.float32


def _make_sc_scatter(layer):
    mesh = plsc.VectorSubcoreMesh(core_axis_name="c", subcore_axis_name="s")

    @functools.partial(
        pl.kernel,
        out_type=jax.ShapeDtypeStruct((NC, N_PAD, H), _f32),
        mesh=mesh,
        scratch_types=[
            pltpu.VMEM((CHUNK,), jnp.int32),
            pltpu.VMEM((CHUNK,), jnp.int32),
            pltpu.VMEM((CHUNK,), jnp.int32),
            pltpu.VMEM((CHUNK, H), _f32),
            pltpu.VMEM((CHUNK, H), _f32),
            pltpu.VMEM((CHUNK, H), _f32),
            pltpu.VMEM((CHUNK, H), _f32),
            pltpu.VMEM((RB_COPY, H), _f32),
            pltpu.VMEM_SHARED((N_PAD, H), _f32),
            pltpu.SemaphoreType.DMA,
            pltpu.SemaphoreType.DMA,
            pltpu.SemaphoreType.DMA,
            pltpu.SemaphoreType.DMA,
        ],
    )
    def sc_scatter(ps, pd, q, src3, dst3, out,
                   idx_s, idx_d0, idx_d1, ba, bb, bq, bres, stage, acc,
                   sem_a, sem_b, sem_q, sem_sc):
        cid = lax.axis_index("c")
        sid = lax.axis_index("s")
        wid = cid * NS + sid
        row0 = sid * ROWS_PER_TILE

        def zrow(r, carry):
            for g in range(LG):
                stage[r, pl.ds(g * 16, 16)] = jnp.zeros((16,), _f32)
            return carry
        lax.fori_loop(0, RB_COPY, zrow, 0)

        def zcp(t, carry):
            pltpu.sync_copy(stage, acc.at[pl.ds(row0 + t * RB_COPY, RB_COPY)])
            return carry
        lax.fori_loop(0, ROWS_PER_TILE // RB_COPY, zcp, 0)
        plsc.subcore_barrier()

        base0 = wid * (CPT * CHUNK)

        pltpu.sync_copy(src3.at[wid, 0], idx_s)
        pltpu.sync_copy(dst3.at[wid, 0], idx_d0)
        pltpu.async_copy(ps.at[idx_s], ba, sem_a)
        pltpu.async_copy(pd.at[idx_d0], bb, sem_b)
        pltpu.async_copy(q.at[layer, pl.ds(base0, CHUNK)], bq, sem_q)

        def body(j, carry):
            pltpu.make_async_copy(ps.at[idx_s], ba, sem_a).wait()
            pltpu.make_async_copy(pd.at[idx_d0], bb, sem_b).wait()
            pltpu.make_async_copy(
                q.at[layer, pl.ds(base0 + j * CHUNK, CHUNK)], bq,
                sem_q).wait()

            @pl.when(j > 0)
            def _wait_prev_scatter():
                pltpu.make_async_copy(bres, acc.at[idx_d0], sem_sc).wait()

            def crow(r, c2):
                for g in range(LG):
                    s = pl.ds(g * 16, 16)
                    bres[r, s] = jnp.maximum(ba[r, s] + bb[r, s] + bq[r, s],
                                             0.0)
                return c2
            lax.fori_loop(0, CHUNK, crow, 0)

            @pl.when(j < CPT - 1)
            def _prefetch_next():
                pltpu.sync_copy(src3.at[wid, j + 1], idx_s)
                pltpu.async_copy(ps.at[idx_s], ba, sem_a)
                pltpu.async_copy(
                    q.at[layer, pl.ds(base0 + (j + 1) * CHUNK, CHUNK)], bq,
                    sem_q)

            @pl.when(jnp.logical_and(j < CPT - 1, j % 2 == 0))
            def _pref_even():
                pltpu.sync_copy(dst3.at[wid, j + 1], idx_d1)
                pltpu.async_copy(pd.at[idx_d1], bb, sem_b)

            @pl.when(jnp.logical_and(j < CPT - 1, j % 2 == 1))
            def _pref_odd():
                pltpu.sync_copy(dst3.at[wid, j + 1], idx_d0)
                pltpu.async_copy(pd.at[idx_d0], bb, sem_b)

            @pl.when(j % 2 == 0)
            def _scatter_even():
                pltpu.async_copy(bres, acc.at[idx_d0], sem_sc, add=True)

            @pl.when(j % 2 == 1)
            def _scatter_odd():
                pltpu.async_copy(bres, acc.at[idx_d1], sem_sc, add=True)
            return carry
        lax.fori_loop(0, CPT, body, 0)
        pltpu.make_async_copy(bres, acc.at[idx_d0], sem_sc).wait()
        plsc.subcore_barrier()

        def readback(t, carry):
            r = row0 + t * RB_COPY
            pltpu.sync_copy(acc.at[pl.ds(r, RB_COPY)], stage)
            pltpu.sync_copy(stage, out.at[cid, pl.ds(r, RB_COPY)])
            return carry
        lax.fori_loop(0, ROWS_PER_TILE // RB_COPY, readback, 0)

    return sc_scatter


def _make_sc_degree():
    mesh = plsc.VectorSubcoreMesh(core_axis_name="c", subcore_axis_name="s")

    @functools.partial(
        pl.kernel,
        out_type=jax.ShapeDtypeStruct((NC, N_PAD, DEG_W), _f32),
        mesh=mesh,
        scratch_types=[
            pltpu.VMEM((CHUNK,), jnp.int32),
            pltpu.VMEM((CHUNK, DEG_W), _f32),
            pltpu.VMEM((RB_COPY, DEG_W), _f32),
            pltpu.VMEM_SHARED((N_PAD, DEG_W), _f32),
        ],
    )
    def sc_degree(dst3, out, idx_d, ones, stage, acc):
        cid = lax.axis_index("c")
        sid = lax.axis_index("s")
        wid = cid * NS + sid
        row0 = sid * ROWS_PER_TILE

        def fill(r, carry):
            ones[r, pl.ds(0, 16)] = jnp.ones((16,), _f32)
            stage[r, pl.ds(0, 16)] = jnp.zeros((16,), _f32)
            return carry
        lax.fori_loop(0, RB_COPY, fill, 0)

        def zcp(t, carry):
            pltpu.sync_copy(stage, acc.at[pl.ds(row0 + t * RB_COPY, RB_COPY)])
            return carry
        lax.fori_loop(0, ROWS_PER_TILE // RB_COPY, zcp, 0)
        plsc.subcore_barrier()

        def body(j, carry):
            pltpu.sync_copy(dst3.at[wid, j], idx_d)
            pltpu.sync_copy(ones, acc.at[idx_d], add=True)
            return carry
        lax.fori_loop(0, CPT, body, 0)
        plsc.subcore_barrier()

        def readback(t, carry):
            r = row0 + t * RB_COPY
            pltpu.sync_copy(acc.at[pl.ds(r, RB_COPY)], stage)
            pltpu.sync_copy(stage, out.at[cid, pl.ds(r, RB_COPY)])
            return carry
        lax.fori_loop(0, ROWS_PER_TILE // RB_COPY, readback, 0)

    return sc_degree


RB_N = 2000
GRID_N = N // RB_N
EB = 2048
GRID_E = E_PAD // EB


def _dot(a, b):
    return jax.lax.dot_general(a, b, (((1,), (0,)), ((), ())),
                               preferred_element_type=_f32)


def _dot_hi(a, b):
    return jax.lax.dot_general(a, b, (((1,), (0,)), ((), ())),
                               preferred_element_type=_f32,
                               precision=jax.lax.Precision.HIGHEST)


def _tc_node_encode(x, neW0, neb0, neW1, neb1, w0s, w0d, b0):
    def body(x_ref, w0_ref, bb0_ref, w1_ref, bb1_ref, ws_ref, wd_ref, be_ref,
             h_ref, ps_ref, pd_ref):
        hb = _dot(jax.nn.relu(_dot(x_ref[...], w0_ref[...]) + bb0_ref[...]),
                  w1_ref[...]) + bb1_ref[...]
        h_ref[...] = hb
        ps_ref[...] = _dot(hb, ws_ref[...])
        pd_ref[...] = _dot(hb, wd_ref[...]) + be_ref[...]

    full = lambda r, c: pl.BlockSpec((r, c), lambda i: (0, 0))
    return pl.pallas_call(
        body,
        grid=(GRID_N,),
        in_specs=[
            pl.BlockSpec((RB_N, NODE_IN), lambda i: (i, 0)),
            full(NODE_IN, H), full(1, H), full(H, H), full(1, H),
            full(H, H), full(H, H), full(1, H),
        ],
        out_specs=[
            pl.BlockSpec((RB_N, H), lambda i: (i, 0)),
            pl.BlockSpec((RB_N, H), lambda i: (i, 0)),
            pl.BlockSpec((RB_N, H), lambda i: (i, 0)),
        ],
        out_shape=[
            jax.ShapeDtypeStruct((N, H), _f32),
            jax.ShapeDtypeStruct((N, H), _f32),
            jax.ShapeDtypeStruct((N, H), _f32),
        ],
    )(x, neW0, neb0, neW1, neb1, w0s, w0d, b0)


def _tc_edge_q(ea, eeW0, eeb0, eeW1, eeb1, w0e_all):
    def body(ea_ref, w0_ref, b0_ref, w1_ref, b1_ref, we_ref, q_ref):
        eb = _dot(jax.nn.relu(_dot(ea_ref[...], w0_ref[...]) + b0_ref[...]),
                  w1_ref[...]) + b1_ref[...]
        for l in range(L):
            q_ref[l] = _dot(eb, we_ref[l])

    full = lambda r, c: pl.BlockSpec((r, c), lambda i: (0, 0))
    return pl.pallas_call(
        body,
        grid=(GRID_E,),
        in_specs=[
            pl.BlockSpec((EB, EDGE_IN), lambda i: (i, 0)),
            full(EDGE_IN, H), full(1, H), full(H, H), full(1, H),
            pl.BlockSpec((L, H, H), lambda i: (0, 0, 0)),
        ],
        out_specs=pl.BlockSpec((L, EB, H), lambda i: (0, i, 0)),
        out_shape=jax.ShapeDtypeStruct((L, E_PAD, H), _f32),
    )(ea, eeW0, eeb0, eeW1, eeb1, w0e_all)


def _tc_layer_update(h, S, D, eW1, eb1, nW0h, nW0a, nb0, nW1, nb1,
                     w0s_n, w0d_n, b0_n):
    def body(h_ref, s_ref, d_ref, w1_ref, b1_ref, wh_ref, wa_ref, bb0_ref,
             nw1_ref, nb1_ref, ws_ref, wd_ref, be_ref,
             h_out, ps_out, pd_out):
        hb = h_ref[...]
        ssum = s_ref[0] + s_ref[1]
        deg = d_ref[0, :, 0:1] + d_ref[1, :, 0:1]
        agg = _dot(ssum, w1_ref[...]) + deg * b1_ref[...]
        t = jax.nn.relu(_dot(hb, wh_ref[...]) + _dot(agg, wa_ref[...])
                        + bb0_ref[...])
        hn = hb + _dot(t, nw1_ref[...]) + nb1_ref[...]
        h_out[...] = hn
        ps_out[...] = _dot(hn, ws_ref[...])
        pd_out[...] = _dot(hn, wd_ref[...]) + be_ref[...]

    full = lambda r, c: pl.BlockSpec((r, c), lambda i: (0, 0))
    return pl.pallas_call(
        body,
        grid=(GRID_N,),
        in_specs=[
            pl.BlockSpec((RB_N, H), lambda i: (i, 0)),
            pl.BlockSpec((NC, RB_N, H), lambda i: (0, i, 0)),
            pl.BlockSpec((NC, RB_N, DEG_W), lambda i: (0, i, 0)),
            full(H, H), full(1, H), full(H, H), full(H, H), full(1, H),
            full(H, H), full(1, H), full(H, H), full(H, H), full(1, H),
        ],
        out_specs=[
            pl.BlockSpec((RB_N, H), lambda i: (i, 0)),
            pl.BlockSpec((RB_N, H), lambda i: (i, 0)),
            pl.BlockSpec((RB_N, H), lambda i: (i, 0)),
        ],
        out_shape=[
            jax.ShapeDtypeStruct((N, H), _f32),
            jax.ShapeDtypeStruct((N, H), _f32),
            jax.ShapeDtypeStruct((N, H), _f32),
        ],
    )(h, S, D, eW1, eb1, nW0h, nW0a, nb0, nW1, nb1, w0s_n, w0d_n, b0_n)


def _tc_final(h, S, D, eW1, eb1, nW0h, nW0a, nb0, nW1, nb1,
              deW0, deb0, deW1, deb1, deW2, deb2, bc_disp, bc_rot):
    def body(h_ref, s_ref, d_ref, w1_ref, b1_ref, wh_ref, wa_ref, bb0_ref,
             nw1_ref, nb1_ref, dw0_ref, db0_ref, dw1_ref, db1_ref,
             dw2_ref, db2_ref, bcd_ref, bcr_ref, out_ref):
        hb = h_ref[...]
        ssum = s_ref[0] + s_ref[1]
        deg = d_ref[0, :, 0:1] + d_ref[1, :, 0:1]
        agg = _dot(ssum, w1_ref[...]) + deg * b1_ref[...]
        t = jax.nn.relu(_dot(hb, wh_ref[...]) + _dot(agg, wa_ref[...])
                        + bb0_ref[...])
        hn = hb + _dot(t, nw1_ref[...]) + nb1_ref[...]
        z = jax.nn.relu(_dot(hn, dw0_ref[...]) + db0_ref[...])
        z = jax.nn.relu(_dot(z, dw1_ref[...]) + db1_ref[...])
        pred = _dot(z, dw2_ref[...]) + db2_ref[...]
        mask = jnp.concatenate([1.0 - bcd_ref[...], 1.0 - bcr_ref[...]],
                               axis=1)
        out_ref[...] = pred * mask

    full = lambda r, c: pl.BlockSpec((r, c), lambda i: (0, 0))
    return pl.pallas_call(
        body,
        grid=(GRID_N,),
        in_specs=[
            pl.BlockSpec((RB_N, H), lambda i: (i, 0)),
            pl.BlockSpec((NC, RB_N, H), lambda i: (0, i, 0)),
            pl.BlockSpec((NC, RB_N, DEG_W), lambda i: (0, i, 0)),
            full(H, H), full(1, H), full(H, H), full(H, H), full(1, H),
            full(H, H), full(1, H),
            full(H, H), full(1, H), full(H, 64), full(1, 64),
            full(64, OUT), full(1, OUT),
            pl.BlockSpec((RB_N, 2), lambda i: (i, 0)),
            pl.BlockSpec((RB_N, 1), lambda i: (i, 0)),
        ],
        out_specs=pl.BlockSpec((RB_N, OUT), lambda i: (i, 0)),
        out_shape=jax.ShapeDtypeStruct((N, OUT), _f32),
    )(h, S, D, eW1, eb1, nW0h, nW0a, nb0, nW1, nb1,
      deW0, deb0, deW1, deb1, deW2, deb2, bc_disp, bc_rot)


_SC_SCATTER = [_make_sc_scatter(l) for l in range(L)]
_SC_DEGREE = _make_sc_degree()


def kernel(x, edge_index, edge_attr, bc_disp, bc_rot, params):
    p = params
    r1 = lambda v: v.reshape(1, -1)

    src = edge_index[0]
    dst = edge_index[1]
    pad = E_PAD - E
    src_p = jnp.concatenate([src, jnp.zeros((pad,), jnp.int32)])
    dst_p = jnp.concatenate([dst, jnp.full((pad,), N, jnp.int32)])
    src_p = src_p.reshape(NW, CPT, CHUNK)
    dst_p = dst_p.reshape(NW, CPT, CHUNK)
    ea_p = jnp.concatenate([edge_attr, jnp.zeros((pad, EDGE_IN), _f32)])

    w0s = [p['mp%d_eW0' % i][:H] for i in range(L)]
    w0d = [p['mp%d_eW0' % i][H:2 * H] for i in range(L)]
    w0e_all = jnp.stack([p['mp%d_eW0' % i][2 * H:] for i in range(L)])

    h, ps, pd = _tc_node_encode(
        x, p['ne_W0'], r1(p['ne_b0']), p['ne_W1'], r1(p['ne_b1']),
        w0s[0], w0d[0], r1(p['mp0_eb0']))
    q_all = _tc_edge_q(ea_p, p['ee_W0'], r1(p['ee_b0']),
                       p['ee_W1'], r1(p['ee_b1']), w0e_all)
    D = _SC_DEGREE(dst_p)

    for i in range(L):
        S = _SC_SCATTER[i](ps, pd, q_all, src_p, dst_p)
        args = (p['mp%d_eW1' % i], r1(p['mp%d_eb1' % i]),
                p['mp%d_nW0' % i][:H], p['mp%d_nW0' % i][H:],
                r1(p['mp%d_nb0' % i]), p['mp%d_nW1' % i],
                r1(p['mp%d_nb1' % i]))
        if i < L - 1:
            h, ps, pd = _tc_layer_update(
                h, S, D, *args,
                w0s[i + 1], w0d[i + 1], r1(p['mp%d_eb0' % (i + 1)]))
        else:
            pred = _tc_final(
                h, S, D, *args,
                p['de_W0'], r1(p['de_b0']), p['de_W1'], r1(p['de_b1']),
                p['de_W2'], r1(p['de_b2']), bc_disp, bc_rot)
    return pred


# speedup vs baseline: 2.5106x; 1.0576x over previous
import functools

import jax
import jax.numpy as jnp
from jax import lax
from jax.experimental import pallas as pl
from jax.experimental.pallas import tpu as pltpu
from jax.experimental.pallas import tpu_sc as plsc

N = 10000
E = 160000
NODE_IN = 10
EDGE_IN = 7
H = 128
L = 6
OUT = 3

NC = 2
NS = 16
NW = NC * NS
CHUNK = 64
CPT = 80
E_PAD = NW * CPT * CHUNK
N_PAD = 10240
ROWS_PER_TILE = N_PAD // NS
RB_COPY = 32
DEG_W = 16
LG = H // 16

_f32 = jnp.float32


def _make_sc_scatter(layer):
    mesh = plsc.VectorSubcoreMesh(core_axis_name="c", subcore_axis_name="s")

    @functools.partial(
        pl.kernel,
        out_type=jax.ShapeDtypeStruct((NC, N_PAD, H), _f32),
        mesh=mesh,
        scratch_types=[
            pltpu.VMEM((CPT, CHUNK), jnp.int32),
            pltpu.VMEM((CHUNK,), jnp.int32),
            pltpu.VMEM((CHUNK,), jnp.int32),
            pltpu.VMEM((CHUNK, H), _f32),
            pltpu.VMEM((CHUNK, H), _f32),
            pltpu.VMEM((CHUNK, H), _f32),
            pltpu.VMEM((CHUNK, H), _f32),
            pltpu.VMEM((RB_COPY, H), _f32),
            pltpu.VMEM_SHARED((N_PAD, H), _f32),
            pltpu.SemaphoreType.DMA,
            pltpu.SemaphoreType.DMA,
            pltpu.SemaphoreType.DMA,
            pltpu.SemaphoreType.DMA,
        ],
    )
    def sc_scatter(ps, pd, q, src3, dst3, out,
                   idx_s, idx_d0, idx_d1, ba, bb, bq, bres, stage, acc,
                   sem_a, sem_b, sem_q, sem_sc):
        cid = lax.axis_index("c")
        sid = lax.axis_index("s")
        wid = cid * NS + sid
        row0 = sid * ROWS_PER_TILE

        def zrow(r, carry):
            for g in range(LG):
                stage[r, pl.ds(g * 16, 16)] = jnp.zeros((16,), _f32)
            return carry
        lax.fori_loop(0, RB_COPY, zrow, 0)

        def zcp(t, carry):
            pltpu.sync_copy(stage, acc.at[pl.ds(row0 + t * RB_COPY, RB_COPY)])
            return carry
        lax.fori_loop(0, ROWS_PER_TILE // RB_COPY, zcp, 0)
        plsc.subcore_barrier()

        base0 = wid * (CPT * CHUNK)

        pltpu.sync_copy(src3.at[wid], idx_s)
        pltpu.sync_copy(dst3.at[wid, 0], idx_d0)
        pltpu.async_copy(ps.at[idx_s.at[0]], ba, sem_a)
        pltpu.async_copy(pd.at[idx_d0], bb, sem_b)
        pltpu.async_copy(q.at[layer, pl.ds(base0, CHUNK)], bq, sem_q)

        def body(j, carry):
            pltpu.make_async_copy(ps.at[idx_s.at[j]], ba, sem_a).wait()
            pltpu.make_async_copy(pd.at[idx_d0], bb, sem_b).wait()
            pltpu.make_async_copy(
                q.at[layer, pl.ds(base0 + j * CHUNK, CHUNK)], bq,
                sem_q).wait()

            @pl.when(j > 0)
            def _wait_prev_scatter():
                pltpu.make_async_copy(bres, acc.at[idx_d0], sem_sc).wait()

            def crow(r, c2):
                for g in range(LG):
                    s = pl.ds(g * 16, 16)
                    bres[r, s] = jnp.maximum(ba[r, s] + bb[r, s] + bq[r, s],
                                             0.0)
                return c2
            lax.fori_loop(0, CHUNK, crow, 0)

            @pl.when(j < CPT - 1)
            def _prefetch_next():
                pltpu.async_copy(ps.at[idx_s.at[j + 1]], ba, sem_a)
                pltpu.async_copy(
                    q.at[layer, pl.ds(base0 + (j + 1) * CHUNK, CHUNK)], bq,
                    sem_q)

            @pl.when(jnp.logical_and(j < CPT - 1, j % 2 == 0))
            def _pref_even():
                pltpu.sync_copy(dst3.at[wid, j + 1], idx_d1)
                pltpu.async_copy(pd.at[idx_d1], bb, sem_b)

            @pl.when(jnp.logical_and(j < CPT - 1, j % 2 == 1))
            def _pref_odd():
                pltpu.sync_copy(dst3.at[wid, j + 1], idx_d0)
                pltpu.async_copy(pd.at[idx_d0], bb, sem_b)

            @pl.when(j % 2 == 0)
            def _scatter_even():
                pltpu.async_copy(bres, acc.at[idx_d0], sem_sc, add=True)

            @pl.when(j % 2 == 1)
            def _scatter_odd():
                pltpu.async_copy(bres, acc.at[idx_d1], sem_sc, add=True)
            return carry
        lax.fori_loop(0, CPT, body, 0)
        pltpu.make_async_copy(bres, acc.at[idx_d0], sem_sc).wait()
        plsc.subcore_barrier()

        def readback(t, carry):
            r = row0 + t * RB_COPY
            pltpu.sync_copy(acc.at[pl.ds(r, RB_COPY)], stage)
            pltpu.sync_copy(stage, out.at[cid, pl.ds(r, RB_COPY)])
            return carry
        lax.fori_loop(0, ROWS_PER_TILE // RB_COPY, readback, 0)

    return sc_scatter


def _make_sc_degree():
    mesh = plsc.VectorSubcoreMesh(core_axis_name="c", subcore_axis_name="s")

    @functools.partial(
        pl.kernel,
        out_type=jax.ShapeDtypeStruct((NC, N_PAD, DEG_W), _f32),
        mesh=mesh,
        scratch_types=[
            pltpu.VMEM((CHUNK,), jnp.int32),
            pltpu.VMEM((CHUNK, DEG_W), _f32),
            pltpu.VMEM((RB_COPY, DEG_W), _f32),
            pltpu.VMEM_SHARED((N_PAD, DEG_W), _f32),
        ],
    )
    def sc_degree(dst3, out, idx_d, ones, stage, acc):
        cid = lax.axis_index("c")
        sid = lax.axis_index("s")
        wid = cid * NS + sid
        row0 = sid * ROWS_PER_TILE

        def fill(r, carry):
            ones[r, pl.ds(0, 16)] = jnp.ones((16,), _f32)
            stage[r, pl.ds(0, 16)] = jnp.zeros((16,), _f32)
            return carry
        lax.fori_loop(0, RB_COPY, fill, 0)

        def zcp(t, carry):
            pltpu.sync_copy(stage, acc.at[pl.ds(row0 + t * RB_COPY, RB_COPY)])
            return carry
        lax.fori_loop(0, ROWS_PER_TILE // RB_COPY, zcp, 0)
        plsc.subcore_barrier()

        def body(j, carry):
            pltpu.sync_copy(dst3.at[wid, j], idx_d)
            pltpu.sync_copy(ones, acc.at[idx_d], add=True)
            return carry
        lax.fori_loop(0, CPT, body, 0)
        plsc.subcore_barrier()

        def readback(t, carry):
            r = row0 + t * RB_COPY
            pltpu.sync_copy(acc.at[pl.ds(r, RB_COPY)], stage)
            pltpu.sync_copy(stage, out.at[cid, pl.ds(r, RB_COPY)])
            return carry
        lax.fori_loop(0, ROWS_PER_TILE // RB_COPY, readback, 0)

    return sc_degree


RB_N = 2000
GRID_N = N // RB_N
EB = 2048
GRID_E = E_PAD // EB


def _dot(a, b):
    return jax.lax.dot_general(a, b, (((1,), (0,)), ((), ())),
                               preferred_element_type=_f32)


def _dot_hi(a, b):
    return jax.lax.dot_general(a, b, (((1,), (0,)), ((), ())),
                               preferred_element_type=_f32,
                               precision=jax.lax.Precision.HIGHEST)


def _tc_node_encode(x, neW0, neb0, neW1, neb1, w0s, w0d, b0):
    def body(x_ref, w0_ref, bb0_ref, w1_ref, bb1_ref, ws_ref, wd_ref, be_ref,
             h_ref, ps_ref, pd_ref):
        hb = _dot(jax.nn.relu(_dot(x_ref[...], w0_ref[...]) + bb0_ref[...]),
                  w1_ref[...]) + bb1_ref[...]
        h_ref[...] = hb
        ps_ref[...] = _dot(hb, ws_ref[...])
        pd_ref[...] = _dot(hb, wd_ref[...]) + be_ref[...]

    full = lambda r, c: pl.BlockSpec((r, c), lambda i: (0, 0))
    return pl.pallas_call(
        body,
        grid=(GRID_N,),
        in_specs=[
            pl.BlockSpec((RB_N, NODE_IN), lambda i: (i, 0)),
            full(NODE_IN, H), full(1, H), full(H, H), full(1, H),
            full(H, H), full(H, H), full(1, H),
        ],
        out_specs=[
            pl.BlockSpec((RB_N, H), lambda i: (i, 0)),
            pl.BlockSpec((RB_N, H), lambda i: (i, 0)),
            pl.BlockSpec((RB_N, H), lambda i: (i, 0)),
        ],
        out_shape=[
            jax.ShapeDtypeStruct((N, H), _f32),
            jax.ShapeDtypeStruct((N, H), _f32),
            jax.ShapeDtypeStruct((N, H), _f32),
        ],
    )(x, neW0, neb0, neW1, neb1, w0s, w0d, b0)


def _tc_edge_q(ea, eeW0, eeb0, eeW1, eeb1, w0e_all):
    def body(ea_ref, w0_ref, b0_ref, w1_ref, b1_ref, we_ref, q_ref):
        eb = _dot(jax.nn.relu(_dot(ea_ref[...], w0_ref[...]) + b0_ref[...]),
                  w1_ref[...]) + b1_ref[...]
        for l in range(L):
            q_ref[l] = _dot(eb, we_ref[l])

    full = lambda r, c: pl.BlockSpec((r, c), lambda i: (0, 0))
    return pl.pallas_call(
        body,
        grid=(GRID_E,),
        in_specs=[
            pl.BlockSpec((EB, EDGE_IN), lambda i: (i, 0)),
            full(EDGE_IN, H), full(1, H), full(H, H), full(1, H),
            pl.BlockSpec((L, H, H), lambda i: (0, 0, 0)),
        ],
        out_specs=pl.BlockSpec((L, EB, H), lambda i: (0, i, 0)),
        out_shape=jax.ShapeDtypeStruct((L, E_PAD, H), _f32),
    )(ea, eeW0, eeb0, eeW1, eeb1, w0e_all)


def _tc_layer_update(h, S, D, eW1, eb1, nW0h, nW0a, nb0, nW1, nb1,
                     w0s_n, w0d_n, b0_n):
    def body(h_ref, s_ref, d_ref, w1_ref, b1_ref, wh_ref, wa_ref, bb0_ref,
             nw1_ref, nb1_ref, ws_ref, wd_ref, be_ref,
             h_out, ps_out, pd_out):
        hb = h_ref[...]
        ssum = s_ref[0] + s_ref[1]
        deg = d_ref[0, :, 0:1] + d_ref[1, :, 0:1]
        agg = _dot(ssum, w1_ref[...]) + deg * b1_ref[...]
        t = jax.nn.relu(_dot(hb, wh_ref[...]) + _dot(agg, wa_ref[...])
                        + bb0_ref[...])
        hn = hb + _dot(t, nw1_ref[...]) + nb1_ref[...]
        h_out[...] = hn
        ps_out[...] = _dot(hn, ws_ref[...])
        pd_out[...] = _dot(hn, wd_ref[...]) + be_ref[...]

    full = lambda r, c: pl.BlockSpec((r, c), lambda i: (0, 0))
    return pl.pallas_call(
        body,
        grid=(GRID_N,),
        in_specs=[
            pl.BlockSpec((RB_N, H), lambda i: (i, 0)),
            pl.BlockSpec((NC, RB_N, H), lambda i: (0, i, 0)),
            pl.BlockSpec((NC, RB_N, DEG_W), lambda i: (0, i, 0)),
            full(H, H), full(1, H), full(H, H), full(H, H), full(1, H),
            full(H, H), full(1, H), full(H, H), full(H, H), full(1, H),
        ],
        out_specs=[
            pl.BlockSpec((RB_N, H), lambda i: (i, 0)),
            pl.BlockSpec((RB_N, H), lambda i: (i, 0)),
            pl.BlockSpec((RB_N, H), lambda i: (i, 0)),
        ],
        out_shape=[
            jax.ShapeDtypeStruct((N, H), _f32),
            jax.ShapeDtypeStruct((N, H), _f32),
            jax.ShapeDtypeStruct((N, H), _f32),
        ],
    )(h, S, D, eW1, eb1, nW0h, nW0a, nb0, nW1, nb1, w0s_n, w0d_n, b0_n)


def _tc_final(h, S, D, eW1, eb1, nW0h, nW0a, nb0, nW1, nb1,
              deW0, deb0, deW1, deb1, deW2, deb2, bc_disp, bc_rot):
    def body(h_ref, s_ref, d_ref, w1_ref, b1_ref, wh_ref, wa_ref, bb0_ref,
             nw1_ref, nb1_ref, dw0_ref, db0_ref, dw1_ref, db1_ref,
             dw2_ref, db2_ref, bcd_ref, bcr_ref, out_ref):
        hb = h_ref[...]
        ssum = s_ref[0] + s_ref[1]
        deg = d_ref[0, :, 0:1] + d_ref[1, :, 0:1]
        agg = _dot(ssum, w1_ref[...]) + deg * b1_ref[...]
        t = jax.nn.relu(_dot(hb, wh_ref[...]) + _dot(agg, wa_ref[...])
                        + bb0_ref[...])
        hn = hb + _dot(t, nw1_ref[...]) + nb1_ref[...]
        z = jax.nn.relu(_dot(hn, dw0_ref[...]) + db0_ref[...])
        z = jax.nn.relu(_dot(z, dw1_ref[...]) + db1_ref[...])
        pred = _dot(z, dw2_ref[...]) + db2_ref[...]
        mask = jnp.concatenate([1.0 - bcd_ref[...], 1.0 - bcr_ref[...]],
                               axis=1)
        out_ref[...] = pred * mask

    full = lambda r, c: pl.BlockSpec((r, c), lambda i: (0, 0))
    return pl.pallas_call(
        body,
        grid=(GRID_N,),
        in_specs=[
            pl.BlockSpec((RB_N, H), lambda i: (i, 0)),
            pl.BlockSpec((NC, RB_N, H), lambda i: (0, i, 0)),
            pl.BlockSpec((NC, RB_N, DEG_W), lambda i: (0, i, 0)),
            full(H, H), full(1, H), full(H, H), full(H, H), full(1, H),
            full(H, H), full(1, H),
            full(H, H), full(1, H), full(H, 64), full(1, 64),
            full(64, OUT), full(1, OUT),
            pl.BlockSpec((RB_N, 2), lambda i: (i, 0)),
            pl.BlockSpec((RB_N, 1), lambda i: (i, 0)),
        ],
        out_specs=pl.BlockSpec((RB_N, OUT), lambda i: (i, 0)),
        out_shape=jax.ShapeDtypeStruct((N, OUT), _f32),
    )(h, S, D, eW1, eb1, nW0h, nW0a, nb0, nW1, nb1,
      deW0, deb0, deW1, deb1, deW2, deb2, bc_disp, bc_rot)


_SC_SCATTER = [_make_sc_scatter(l) for l in range(L)]
_SC_DEGREE = _make_sc_degree()


def kernel(x, edge_index, edge_attr, bc_disp, bc_rot, params):
    p = params
    r1 = lambda v: v.reshape(1, -1)

    src = edge_index[0]
    dst = edge_index[1]
    pad = E_PAD - E
    src_p = jnp.concatenate([src, jnp.zeros((pad,), jnp.int32)])
    dst_p = jnp.concatenate([dst, jnp.full((pad,), N, jnp.int32)])
    src_p = src_p.reshape(NW, CPT, CHUNK)
    dst_p = dst_p.reshape(NW, CPT, CHUNK)
    ea_p = jnp.concatenate([edge_attr, jnp.zeros((pad, EDGE_IN), _f32)])

    w0s = [p['mp%d_eW0' % i][:H] for i in range(L)]
    w0d = [p['mp%d_eW0' % i][H:2 * H] for i in range(L)]
    w0e_all = jnp.stack([p['mp%d_eW0' % i][2 * H:] for i in range(L)])

    h, ps, pd = _tc_node_encode(
        x, p['ne_W0'], r1(p['ne_b0']), p['ne_W1'], r1(p['ne_b1']),
        w0s[0], w0d[0], r1(p['mp0_eb0']))
    q_all = _tc_edge_q(ea_p, p['ee_W0'], r1(p['ee_b0']),
                       p['ee_W1'], r1(p['ee_b1']), w0e_all)
    D = _SC_DEGREE(dst_p)

    for i in range(L):
        S = _SC_SCATTER[i](ps, pd, q_all, src_p, dst_p)
        args = (p['mp%d_eW1' % i], r1(p['mp%d_eb1' % i]),
                p['mp%d_nW0' % i][:H], p['mp%d_nW0' % i][H:],
                r1(p['mp%d_nb0' % i]), p['mp%d_nW1' % i],
                r1(p['mp%d_nb1' % i]))
        if i < L - 1:
            h, ps, pd = _tc_layer_update(
                h, S, D, *args,
                w0s[i + 1], w0d[i + 1], r1(p['mp%d_eb0' % (i + 1)]))
        else:
            pred = _tc_final(
                h, S, D, *args,
                p['de_W0'], r1(p['de_b0']), p['de_W1'], r1(p['de_b1']),
                p['de_W2'], r1(p['de_b2']), bc_disp, bc_rot)
    return pred
